# Initial kernel scaffold; baseline (speedup 1.0000x reference)
import functools

import jax
import jax.numpy as jnp
from jax import lax
from jax.experimental import pallas as pl
from jax.experimental.pallas import tpu as pltpu
from jax.experimental.pallas import tpu_sc as plsc

_info = plsc.get_sparse_core_info()
_NC, _NS, _L = _info.num_cores, _info.num_subcores, _info.num_lanes
_NW = _NC * _NS
_CHUNK = 128
_BPW = 3200
_NB_PAD = _NW * _BPW
_NCHUNK = _BPW // _CHUNK
_NVEC = _BPW // _L


def _sc_energy(coords4, idx0, idx1, b0p, kbp):
    mesh = plsc.VectorSubcoreMesh(core_axis_name="c", subcore_axis_name="s")

    @functools.partial(
        pl.kernel,
        out_type=jax.ShapeDtypeStruct((_NW, _L), jnp.float32),
        mesh=mesh,
        scratch_types=[
            pltpu.VMEM((_BPW,), jnp.int32),
            pltpu.VMEM((_BPW,), jnp.int32),
            pltpu.VMEM((_BPW, 4), jnp.float32),
            pltpu.VMEM((_BPW, 4), jnp.float32),
            pltpu.VMEM((_BPW,), jnp.float32),
            pltpu.VMEM((_BPW,), jnp.float32),
            pltpu.VMEM((_L,), jnp.float32),
            pltpu.SemaphoreType.DMA,
            pltpu.SemaphoreType.DMA,
        ],
    )
    def k(coords_h, i0_h, i1_h, b0_h, kb_h, out_h,
          i0_v, i1_v, ri_v, rj_v, b0_v, kb_v, acc_v, sem_i, sem_j):
        wid = lax.axis_index("s") * _NC + lax.axis_index("c")
        base = wid * _BPW
        pltpu.sync_copy(i0_h.at[pl.ds(base, _BPW)], i0_v)
        pltpu.sync_copy(i1_h.at[pl.ds(base, _BPW)], i1_v)
        pltpu.sync_copy(b0_h.at[pl.ds(base, _BPW)], b0_v)
        pltpu.sync_copy(kb_h.at[pl.ds(base, _BPW)], kb_v)

        def fire(c, carry):
            off = c * _CHUNK
            pltpu.async_copy(coords_h.at[i0_v.at[pl.ds(off, _CHUNK)]],
                             ri_v.at[pl.ds(off, _CHUNK)], sem_i)
            pltpu.async_copy(coords_h.at[i1_v.at[pl.ds(off, _CHUNK)]],
                             rj_v.at[pl.ds(off, _CHUNK)], sem_j)
            return carry

        lax.fori_loop(0, _NCHUNK, fire, 0)
        pltpu.make_async_copy(coords_h.at[pl.ds(0, _BPW)], ri_v, sem_i).wait()
        pltpu.make_async_copy(coords_h.at[pl.ds(0, _BPW)], rj_v, sem_j).wait()

        iota = lax.iota(jnp.int32, _L)
        c0 = jnp.zeros((_L,), jnp.int32)
        c1 = c0 + 1
        c2 = c0 + 2

        def body(t, acc):
            b = t * _L + iota
            xi = plsc.load_gather(ri_v, [b, c0])
            yi = plsc.load_gather(ri_v, [b, c1])
            zi = plsc.load_gather(ri_v, [b, c2])
            xj = plsc.load_gather(rj_v, [b, c0])
            yj = plsc.load_gather(rj_v, [b, c1])
            zj = plsc.load_gather(rj_v, [b, c2])
            dx = xi - xj
            dy = yi - yj
            dz = zi - zj
            s = dx * dx + dy * dy + dz * dz
            r = jnp.sqrt(s)
            off = t * _L
            kb = kb_v[pl.ds(off, _L)]
            d = r - b0_v[pl.ds(off, _L)]
            return acc + (0.5 * kb) * (d * d)

        acc = lax.fori_loop(0, _NVEC, body, jnp.zeros((_L,), jnp.float32))
        acc_v[...] = acc
        pltpu.sync_copy(acc_v, out_h.at[wid])

    return k(coords4, idx0, idx1, b0p, kbp)


def kernel(coords, box, bonds, b0, k_bond):
    del box
    coords4 = jnp.pad(coords, ((0, 0), (0, 1)))
    nb = b0.shape[0]
    pad = _NB_PAD - nb
    idx0 = jnp.pad(bonds[:, 0], (0, pad))
    idx1 = jnp.pad(bonds[:, 1], (0, pad))
    b0p = jnp.pad(b0, (0, pad))
    kbp = jnp.pad(k_bond, (0, pad))
    partials = _sc_energy(coords4, idx0, idx1, b0p, kbp)
    return jnp.sum(partials)


# speedup vs baseline: 1.3737x; 1.3737x over previous
import functools

import jax
import jax.numpy as jnp
from jax import lax
from jax.experimental import pallas as pl
from jax.experimental.pallas import tpu as pltpu
from jax.experimental.pallas import tpu_sc as plsc

_info = plsc.get_sparse_core_info()
_NC, _NS, _L = _info.num_cores, _info.num_subcores, _info.num_lanes
_NW = _NC * _NS
_CHUNK = 128
_BPW = 3200
_NB_PAD = _NW * _BPW
_NCHUNK = _BPW // _CHUNK
_NVEC = _BPW // _L


def _sc_energy(coords4, idx0, idx1, b0p, kbp):
    mesh = plsc.VectorSubcoreMesh(core_axis_name="c", subcore_axis_name="s")

    @functools.partial(
        pl.kernel,
        out_type=jax.ShapeDtypeStruct((_NW, _L), jnp.float32),
        mesh=mesh,
        compiler_params=pltpu.CompilerParams(
            needs_layout_passes=False, use_tc_tiling_on_sc=False),
        scratch_types=[
            pltpu.VMEM((_BPW,), jnp.int32),
            pltpu.VMEM((_BPW,), jnp.int32),
            pltpu.VMEM((_BPW, 8), jnp.float32),
            pltpu.VMEM((_BPW, 8), jnp.float32),
            pltpu.VMEM((_BPW,), jnp.float32),
            pltpu.VMEM((_BPW,), jnp.float32),
            pltpu.VMEM((_L,), jnp.float32),
            pltpu.SemaphoreType.DMA,
            pltpu.SemaphoreType.DMA,
        ],
    )
    def k(coords_h, i0_h, i1_h, b0_h, kb_h, out_h,
          i0_v, i1_v, ri_v, rj_v, b0_v, kb_v, acc_v, sem_i, sem_j):
        wid = lax.axis_index("s") * _NC + lax.axis_index("c")
        base = wid * _BPW
        pltpu.sync_copy(i0_h.at[pl.ds(base, _BPW)], i0_v)
        pltpu.sync_copy(i1_h.at[pl.ds(base, _BPW)], i1_v)
        pltpu.sync_copy(b0_h.at[pl.ds(base, _BPW)], b0_v)
        pltpu.sync_copy(kb_h.at[pl.ds(base, _BPW)], kb_v)

        def fire(c, carry):
            off = c * _CHUNK
            cp_i = pltpu.async_copy(coords_h.at[i0_v.at[pl.ds(off, _CHUNK)]],
                                    ri_v.at[pl.ds(off, _CHUNK)], sem_i)
            cp_j = pltpu.async_copy(coords_h.at[i1_v.at[pl.ds(off, _CHUNK)]],
                                    rj_v.at[pl.ds(off, _CHUNK)], sem_j)
            cp_i.wait()
            cp_j.wait()
            return carry

        lax.fori_loop(0, _NCHUNK, fire, 0)

        iota = lax.iota(jnp.int32, _L)
        c0 = jnp.zeros((_L,), jnp.int32)
        c1 = c0 + 1
        c2 = c0 + 2

        def body(t, acc):
            b = t * _L + iota
            xi = plsc.load_gather(ri_v, [b, c0])
            yi = plsc.load_gather(ri_v, [b, c1])
            zi = plsc.load_gather(ri_v, [b, c2])
            xj = plsc.load_gather(rj_v, [b, c0])
            yj = plsc.load_gather(rj_v, [b, c1])
            zj = plsc.load_gather(rj_v, [b, c2])
            dx = xi - xj
            dy = yi - yj
            dz = zi - zj
            s = dx * dx + dy * dy + dz * dz
            bits = lax.bitcast_convert_type(s, jnp.int32)
            y = lax.bitcast_convert_type(
                jnp.int32(0x5F3759DF) - (bits >> 1), jnp.float32)
            hs = 0.5 * s
            y = y * (1.5 - hs * y * y)
            y = y * (1.5 - hs * y * y)
            y = y * (1.5 - hs * y * y)
            r = s * y
            off = t * _L
            kb = kb_v[pl.ds(off, _L)]
            d = r - b0_v[pl.ds(off, _L)]
            return acc + (0.5 * kb) * (d * d)

        acc = lax.fori_loop(0, _NVEC, body, jnp.zeros((_L,), jnp.float32))
        acc_v[...] = acc
        pltpu.sync_copy(acc_v, out_h.at[wid])

    return k(coords4, idx0, idx1, b0p, kbp)


def kernel(coords, box, bonds, b0, k_bond):
    del box
    coords4 = jnp.pad(coords, ((0, 0), (0, 5)))
    nb = b0.shape[0]
    pad = _NB_PAD - nb
    idx0 = jnp.pad(bonds[:, 0], (0, pad))
    idx1 = jnp.pad(bonds[:, 1], (0, pad))
    b0p = jnp.pad(b0, (0, pad))
    kbp = jnp.pad(k_bond, (0, pad))
    partials = _sc_energy(coords4, idx0, idx1, b0p, kbp)
    return jnp.sum(partials)


# speedup vs baseline: 1.4440x; 1.0512x over previous
import functools

import jax
import jax.numpy as jnp
from jax import lax
from jax.experimental import pallas as pl
from jax.experimental.pallas import tpu as pltpu
from jax.experimental.pallas import tpu_sc as plsc

_info = plsc.get_sparse_core_info()
_NC, _NS, _L = _info.num_cores, _info.num_subcores, _info.num_lanes
_NW = _NC * _NS
_CHUNK = 128
_BPW = 3200
_NB_PAD = _NW * _BPW
_NCHUNK = _BPW // _CHUNK
_NVEC = _BPW // _L


def _sc_energy(coords4, idx0, idx1, b0p, kbp):
    mesh = plsc.VectorSubcoreMesh(core_axis_name="c", subcore_axis_name="s")

    @functools.partial(
        pl.kernel,
        out_type=jax.ShapeDtypeStruct((_NW, _L), jnp.float32),
        mesh=mesh,
        compiler_params=pltpu.CompilerParams(
            needs_layout_passes=False, use_tc_tiling_on_sc=False),
        scratch_types=[
            pltpu.VMEM((_BPW,), jnp.int32),
            pltpu.VMEM((_BPW,), jnp.int32),
            pltpu.VMEM((_BPW, 8), jnp.float32),
            pltpu.VMEM((_BPW, 8), jnp.float32),
            pltpu.VMEM((_BPW,), jnp.float32),
            pltpu.VMEM((_BPW,), jnp.float32),
            pltpu.VMEM((_L,), jnp.float32),
            pltpu.SemaphoreType.DMA,
            pltpu.SemaphoreType.DMA,
        ],
    )
    def k(coords_h, i0_h, i1_h, b0_h, kb_h, out_h,
          i0_v, i1_v, ri_v, rj_v, b0_v, kb_v, acc_v, sem_i, sem_j):
        wid = lax.axis_index("s") * _NC + lax.axis_index("c")
        base = wid * _BPW
        pltpu.sync_copy(i0_h.at[pl.ds(base, _BPW)], i0_v)
        pltpu.sync_copy(i1_h.at[pl.ds(base, _BPW)], i1_v)
        pltpu.sync_copy(b0_h.at[pl.ds(base, _BPW)], b0_v)
        pltpu.sync_copy(kb_h.at[pl.ds(base, _BPW)], kb_v)

        cp_i = pltpu.async_copy(coords_h.at[i0_v], ri_v, sem_i)
        cp_j = pltpu.async_copy(coords_h.at[i1_v], rj_v, sem_j)
        cp_i.wait()
        cp_j.wait()

        iota = lax.iota(jnp.int32, _L)
        c0 = jnp.zeros((_L,), jnp.int32)
        c1 = c0 + 1
        c2 = c0 + 2

        def body(t, acc):
            b = t * _L + iota
            xi = plsc.load_gather(ri_v, [b, c0])
            yi = plsc.load_gather(ri_v, [b, c1])
            zi = plsc.load_gather(ri_v, [b, c2])
            xj = plsc.load_gather(rj_v, [b, c0])
            yj = plsc.load_gather(rj_v, [b, c1])
            zj = plsc.load_gather(rj_v, [b, c2])
            dx = xi - xj
            dy = yi - yj
            dz = zi - zj
            s = dx * dx + dy * dy + dz * dz
            bits = lax.bitcast_convert_type(s, jnp.int32)
            y = lax.bitcast_convert_type(
                jnp.int32(0x5F3759DF) - (bits >> 1), jnp.float32)
            hs = 0.5 * s
            y = y * (1.5 - hs * y * y)
            y = y * (1.5 - hs * y * y)
            y = y * (1.5 - hs * y * y)
            r = s * y
            off = t * _L
            kb = kb_v[pl.ds(off, _L)]
            d = r - b0_v[pl.ds(off, _L)]
            return acc + (0.5 * kb) * (d * d)

        acc = lax.fori_loop(0, _NVEC, body, jnp.zeros((_L,), jnp.float32))
        acc_v[...] = acc
        pltpu.sync_copy(acc_v, out_h.at[wid])

    return k(coords4, idx0, idx1, b0p, kbp)


def kernel(coords, box, bonds, b0, k_bond):
    del box
    coords4 = jnp.pad(coords, ((0, 0), (0, 5)))
    nb = b0.shape[0]
    pad = _NB_PAD - nb
    idx0 = jnp.pad(bonds[:, 0], (0, pad))
    idx1 = jnp.pad(bonds[:, 1], (0, pad))
    b0p = jnp.pad(b0, (0, pad))
    kbp = jnp.pad(k_bond, (0, pad))
    partials = _sc_energy(coords4, idx0, idx1, b0p, kbp)
    return jnp.sum(partials)


# speedup vs baseline: 1.4512x; 1.0050x over previous
import functools

import jax
import jax.numpy as jnp
from jax import lax
from jax.experimental import pallas as pl
from jax.experimental.pallas import tpu as pltpu
from jax.experimental.pallas import tpu_sc as plsc

_info = plsc.get_sparse_core_info()
_NC, _NS, _L = _info.num_cores, _info.num_subcores, _info.num_lanes
_NW = _NC * _NS
_CHUNK = 128
_BPW = 3200
_NB_PAD = _NW * _BPW
_NCHUNK = _BPW // _CHUNK
_NVEC = _BPW // _L


def _sc_energy(coords4, idx0, idx1, b0p, kbp):
    mesh = plsc.VectorSubcoreMesh(core_axis_name="c", subcore_axis_name="s")

    @functools.partial(
        pl.kernel,
        out_type=jax.ShapeDtypeStruct((_NW, _L), jnp.float32),
        mesh=mesh,
        compiler_params=pltpu.CompilerParams(
            needs_layout_passes=False, use_tc_tiling_on_sc=False),
        scratch_types=[
            pltpu.VMEM((_BPW,), jnp.int32),
            pltpu.VMEM((_BPW,), jnp.int32),
            pltpu.VMEM((_BPW, 8), jnp.float32),
            pltpu.VMEM((_BPW, 8), jnp.float32),
            pltpu.VMEM((_BPW,), jnp.float32),
            pltpu.VMEM((_BPW,), jnp.float32),
            pltpu.VMEM((_L,), jnp.float32),
            pltpu.SemaphoreType.DMA,
            pltpu.SemaphoreType.DMA,
        ],
    )
    def k(coords_h, i0_h, i1_h, b0_h, kb_h, out_h,
          i0_v, i1_v, ri_v, rj_v, b0_v, kb_v, acc_v, sem_i, sem_j):
        wid = lax.axis_index("s") * _NC + lax.axis_index("c")
        base = wid * _BPW
        half = _BPW // 2
        pltpu.sync_copy(i0_h.at[pl.ds(base, _BPW)], i0_v)
        pltpu.sync_copy(i1_h.at[pl.ds(base, _BPW)], i1_v)
        cp_ai = pltpu.async_copy(coords_h.at[i0_v.at[pl.ds(0, half)]],
                                 ri_v.at[pl.ds(0, half)], sem_i)
        cp_aj = pltpu.async_copy(coords_h.at[i1_v.at[pl.ds(0, half)]],
                                 rj_v.at[pl.ds(0, half)], sem_i)
        cp_bi = pltpu.async_copy(coords_h.at[i0_v.at[pl.ds(half, half)]],
                                 ri_v.at[pl.ds(half, half)], sem_j)
        cp_bj = pltpu.async_copy(coords_h.at[i1_v.at[pl.ds(half, half)]],
                                 rj_v.at[pl.ds(half, half)], sem_j)
        pltpu.sync_copy(b0_h.at[pl.ds(base, _BPW)], b0_v)
        pltpu.sync_copy(kb_h.at[pl.ds(base, _BPW)], kb_v)

        iota = lax.iota(jnp.int32, _L)
        c0 = jnp.zeros((_L,), jnp.int32)
        c1 = c0 + 1
        c2 = c0 + 2

        def body(t, acc):
            b = t * _L + iota
            xi = plsc.load_gather(ri_v, [b, c0])
            yi = plsc.load_gather(ri_v, [b, c1])
            zi = plsc.load_gather(ri_v, [b, c2])
            xj = plsc.load_gather(rj_v, [b, c0])
            yj = plsc.load_gather(rj_v, [b, c1])
            zj = plsc.load_gather(rj_v, [b, c2])
            dx = xi - xj
            dy = yi - yj
            dz = zi - zj
            s = dx * dx + dy * dy + dz * dz
            bits = lax.bitcast_convert_type(s, jnp.int32)
            y = lax.bitcast_convert_type(
                jnp.int32(0x5F3759DF) - (bits >> 1), jnp.float32)
            hs = 0.5 * s
            y = y * (1.5 - hs * y * y)
            y = y * (1.5 - hs * y * y)
            y = y * (1.5 - hs * y * y)
            r = s * y
            off = t * _L
            kb = kb_v[pl.ds(off, _L)]
            d = r - b0_v[pl.ds(off, _L)]
            return acc + (0.5 * kb) * (d * d)

        cp_ai.wait()
        cp_aj.wait()
        acc = lax.fori_loop(0, _NVEC // 2, body, jnp.zeros((_L,), jnp.float32))
        cp_bi.wait()
        cp_bj.wait()
        acc = lax.fori_loop(_NVEC // 2, _NVEC, body, acc)
        acc_v[...] = acc
        pltpu.sync_copy(acc_v, out_h.at[wid])

    return k(coords4, idx0, idx1, b0p, kbp)


def kernel(coords, box, bonds, b0, k_bond):
    del box
    coords4 = jnp.pad(coords, ((0, 0), (0, 5)))
    nb = b0.shape[0]
    pad = _NB_PAD - nb
    idx0 = jnp.pad(bonds[:, 0], (0, pad))
    idx1 = jnp.pad(bonds[:, 1], (0, pad))
    b0p = jnp.pad(b0, (0, pad))
    kbp = jnp.pad(k_bond, (0, pad))
    partials = _sc_energy(coords4, idx0, idx1, b0p, kbp)
    return jnp.sum(partials)


# speedup vs baseline: 1.4532x; 1.0014x over previous
import functools

import jax
import jax.numpy as jnp
from jax import lax
from jax.experimental import pallas as pl
from jax.experimental.pallas import tpu as pltpu
from jax.experimental.pallas import tpu_sc as plsc

_info = plsc.get_sparse_core_info()
_NC, _NS, _L = _info.num_cores, _info.num_subcores, _info.num_lanes
_NW = _NC * _NS
_CHUNK = 128
_BPW = 3200
_NB_PAD = _NW * _BPW
_NCHUNK = _BPW // _CHUNK
_NVEC = _BPW // _L


def _sc_energy(coords4, idx0, idx1, b0p, kbp):
    mesh = plsc.VectorSubcoreMesh(core_axis_name="c", subcore_axis_name="s")

    @functools.partial(
        pl.kernel,
        out_type=jax.ShapeDtypeStruct((_NW, _L), jnp.float32),
        mesh=mesh,
        compiler_params=pltpu.CompilerParams(
            needs_layout_passes=False, use_tc_tiling_on_sc=False,
            skip_device_barrier=True),
        scratch_types=[
            pltpu.VMEM((_BPW,), jnp.int32),
            pltpu.VMEM((_BPW,), jnp.int32),
            pltpu.VMEM((_BPW, 8), jnp.float32),
            pltpu.VMEM((_BPW, 8), jnp.float32),
            pltpu.VMEM((_BPW,), jnp.float32),
            pltpu.VMEM((_BPW,), jnp.float32),
            pltpu.VMEM((_L,), jnp.float32),
            pltpu.SemaphoreType.DMA,
            pltpu.SemaphoreType.DMA,
        ],
    )
    def k(coords_h, i0_h, i1_h, b0_h, kb_h, out_h,
          i0_v, i1_v, ri_v, rj_v, b0_v, kb_v, acc_v, sem_i, sem_j):
        wid = lax.axis_index("s") * _NC + lax.axis_index("c")
        base = wid * _BPW
        half = _BPW // 2
        pltpu.sync_copy(i0_h.at[pl.ds(base, _BPW)], i0_v)
        pltpu.sync_copy(i1_h.at[pl.ds(base, _BPW)], i1_v)
        cp_ai = pltpu.async_copy(coords_h.at[i0_v.at[pl.ds(0, half)]],
                                 ri_v.at[pl.ds(0, half)], sem_i)
        cp_aj = pltpu.async_copy(coords_h.at[i1_v.at[pl.ds(0, half)]],
                                 rj_v.at[pl.ds(0, half)], sem_i)
        cp_bi = pltpu.async_copy(coords_h.at[i0_v.at[pl.ds(half, half)]],
                                 ri_v.at[pl.ds(half, half)], sem_j)
        cp_bj = pltpu.async_copy(coords_h.at[i1_v.at[pl.ds(half, half)]],
                                 rj_v.at[pl.ds(half, half)], sem_j)
        pltpu.sync_copy(b0_h.at[pl.ds(base, _BPW)], b0_v)
        pltpu.sync_copy(kb_h.at[pl.ds(base, _BPW)], kb_v)

        iota = lax.iota(jnp.int32, _L)
        c0 = jnp.zeros((_L,), jnp.int32)
        c1 = c0 + 1
        c2 = c0 + 2

        def body(t, acc):
            b = t * _L + iota
            xi = plsc.load_gather(ri_v, [b, c0])
            yi = plsc.load_gather(ri_v, [b, c1])
            zi = plsc.load_gather(ri_v, [b, c2])
            xj = plsc.load_gather(rj_v, [b, c0])
            yj = plsc.load_gather(rj_v, [b, c1])
            zj = plsc.load_gather(rj_v, [b, c2])
            dx = xi - xj
            dy = yi - yj
            dz = zi - zj
            s = dx * dx + dy * dy + dz * dz
            bits = lax.bitcast_convert_type(s, jnp.int32)
            y = lax.bitcast_convert_type(
                jnp.int32(0x5F3759DF) - (bits >> 1), jnp.float32)
            hs = 0.5 * s
            y = y * (1.5 - hs * y * y)
            y = y * (1.5 - hs * y * y)
            y = y * (1.5 - hs * y * y)
            r = s * y
            off = t * _L
            kb = kb_v[pl.ds(off, _L)]
            d = r - b0_v[pl.ds(off, _L)]
            return acc + (0.5 * kb) * (d * d)

        cp_ai.wait()
        cp_aj.wait()
        acc = lax.fori_loop(0, _NVEC // 2, body, jnp.zeros((_L,), jnp.float32))
        cp_bi.wait()
        cp_bj.wait()
        acc = lax.fori_loop(_NVEC // 2, _NVEC, body, acc)
        acc_v[...] = acc
        pltpu.sync_copy(acc_v, out_h.at[wid])

    return k(coords4, idx0, idx1, b0p, kbp)


def kernel(coords, box, bonds, b0, k_bond):
    del box
    coords4 = jnp.pad(coords, ((0, 0), (0, 5)))
    nb = b0.shape[0]
    pad = _NB_PAD - nb
    idx0 = jnp.pad(bonds[:, 0], (0, pad))
    idx1 = jnp.pad(bonds[:, 1], (0, pad))
    b0p = jnp.pad(b0, (0, pad))
    kbp = jnp.pad(k_bond, (0, pad))
    partials = _sc_energy(coords4, idx0, idx1, b0p, kbp)
    return jnp.sum(partials)
